# baseline (speedup 1.0000x reference)
import functools

import jax
import jax.numpy as jnp
from jax import lax
from jax.experimental import pallas as pl
from jax.experimental.pallas import tpu as pltpu
from jax.experimental.pallas import tpu_sc as plsc

_B = 16384
_D = 32
_L = 16
_NC = 2
_NS = 16
_BPS = _B // _NS
_CHUNK = 128
_NCHUNK = _BPS // _CHUNK


def _sc_kernel(movies_hbm, table2_hbm, user2_hbm, out_hbm,
               idx_v, idx2_v, rows_v, stage_v, part_sh, sums_v, user_v,
               res_v, sem):
    cid = lax.axis_index("c")
    sid = lax.axis_index("s")
    base = sid * _BPS

    pltpu.sync_copy(movies_hbm.at[pl.ds(base, _BPS)], idx_v)

    descs = []
    for j in range(_NCHUNK):
        for k in range(_CHUNK // _L):
            v = idx_v[pl.ds(j * _CHUNK + k * _L, _L)]
            idx2_v[j, pl.ds(k * _L, _L)] = v * 2 + cid
        descs.append(
            pltpu.async_copy(table2_hbm.at[idx2_v.at[j]],
                             rows_v.at[pl.ds(j * _CHUNK, _CHUNK)], sem))

    accs = [jnp.zeros((_L,), jnp.float32) for _ in range(4)]

    for j in range(_NCHUNK):
        descs[j].wait()

        def red(i, a):
            a0, a1, a2, a3 = a
            b = j * _CHUNK + i * 8
            a0 = a0 + rows_v[b + 0]
            a1 = a1 + rows_v[b + 1]
            a2 = a2 + rows_v[b + 2]
            a3 = a3 + rows_v[b + 3]
            a0 = a0 + rows_v[b + 4]
            a1 = a1 + rows_v[b + 5]
            a2 = a2 + rows_v[b + 6]
            a3 = a3 + rows_v[b + 7]
            return (a0, a1, a2, a3)

        accs = lax.fori_loop(0, _CHUNK // 8, red, tuple(accs))
        accs = list(accs)

    acc = (accs[0] + accs[1]) + (accs[2] + accs[3])

    stage_v[...] = acc
    pltpu.sync_copy(stage_v, part_sh.at[sid])
    plsc.subcore_barrier()

    @pl.when(sid == 0)
    def _():
        pltpu.sync_copy(part_sh, sums_v)
        tot = jnp.zeros((_L,), jnp.float32)
        for i in range(_NS):
            tot = tot + sums_v[i]
        pltpu.sync_copy(user2_hbm.at[cid], user_v)
        res_v[...] = tot * user_v[...]
        pltpu.sync_copy(res_v, out_hbm.at[cid])


@jax.jit
def _run(movies, table2, user2):
    mesh = plsc.VectorSubcoreMesh(core_axis_name="c", subcore_axis_name="s")
    f = functools.partial(
        pl.kernel,
        out_type=jax.ShapeDtypeStruct((_NC, _L), jnp.float32),
        mesh=mesh,
        compiler_params=pltpu.CompilerParams(use_tc_tiling_on_sc=False),
        scratch_types=[
            pltpu.VMEM((_BPS,), jnp.int32),
            pltpu.VMEM((_NCHUNK, _CHUNK), jnp.int32),
            pltpu.VMEM((_BPS, _L), jnp.float32),
            pltpu.VMEM((_L,), jnp.float32),
            pltpu.VMEM_SHARED((_NS, _L), jnp.float32),
            pltpu.VMEM((_NS, _L), jnp.float32),
            pltpu.VMEM((_L,), jnp.float32),
            pltpu.VMEM((_L,), jnp.float32),
            pltpu.SemaphoreType.DMA,
        ],
    )(_sc_kernel)
    return f(movies, table2, user2)


def kernel(users, movies, movie_table, user_table):
    table2 = movie_table.reshape(movie_table.shape[0] * 2, _L)
    user2 = user_table.reshape(_NC, _L)
    movies_i = movies.astype(jnp.int32)
    out2 = _run(movies_i, table2, user2)
    return out2.reshape(1, _D)


# speedup vs baseline: 3.4765x; 3.4765x over previous
import functools

import jax
import jax.numpy as jnp
from jax import lax
from jax.experimental import pallas as pl
from jax.experimental.pallas import tpu as pltpu
from jax.experimental.pallas import tpu_sc as plsc

_V = 1000000
_D = 32
_B = 16384
_L = 16
_NC = 2
_NS = 16
_BPW = _B // (_NC * _NS)
_IC = 128
_NIC = _BPW // _IC

_C = 8192
_NSTEP = -(-_V // _C)
_HV = _NSTEP * _C
_ZW = 10
_ZCH = _V // _ZW
_ZCHUNK = 20000
_ZITER = _ZCH // _ZCHUNK


def _sc_hist(movies_hbm, zeros_hbm, hist_hbm, idx_v, ones_v, buf_v, hist_sh):
    cid = lax.axis_index("c")
    sid = lax.axis_index("s")

    @pl.when(sid < _ZW)
    def _():
        pltpu.sync_copy(zeros_hbm, buf_v)
        for z in range(_ZITER):
            pltpu.sync_copy(
                buf_v, hist_sh.at[pl.ds(sid * _ZCH + z * _ZCHUNK, _ZCHUNK)])

    base = (cid * _NS + sid) * _BPW
    for j in range(_NIC):
        pltpu.sync_copy(movies_hbm.at[pl.ds(base + j * _IC, _IC)],
                        idx_v.at[j])

    for k in range(_IC // _L):
        ones_v[pl.ds(k * _L, _L)] = jnp.full((_L,), 1.0, jnp.float32)

    plsc.subcore_barrier()

    for j in range(_NIC):
        pltpu.sync_copy(ones_v, hist_sh.at[idx_v.at[j]], add=True)

    plsc.subcore_barrier()

    @pl.when(sid < _ZW)
    def _():
        for z in range(_ZITER):
            off = sid * _ZCH + z * _ZCHUNK
            pltpu.sync_copy(hist_sh.at[pl.ds(off, _ZCHUNK)], buf_v)
            pltpu.sync_copy(buf_v, hist_hbm.at[pl.ds(cid * _HV + off, _ZCHUNK)])


def _tc_matvec(tabT_ref, h0_ref, h1_ref, userT_ref, out_ref, acc_ref):
    i = pl.program_id(0)

    @pl.when(i == 0)
    def _():
        acc_ref[...] = jnp.zeros_like(acc_ref)

    h = h0_ref[...] + h1_ref[...]
    prod = tabT_ref[...] * h[None, :]
    cols = i * _C + lax.broadcasted_iota(jnp.int32, (_D, _C), 1)
    prod = jnp.where(cols < _V, prod, 0.0)
    acc_ref[...] += jnp.sum(prod.reshape(_D, _C // 128, 128), axis=1)

    @pl.when(i == _NSTEP - 1)
    def _():
        s = jnp.sum(acc_ref[...], axis=1, keepdims=True)
        out_ref[...] = s * userT_ref[...]


@jax.jit
def _run(movies, movie_table, user_table):
    zeros = jnp.zeros((_ZCHUNK,), jnp.float32)
    mesh = plsc.VectorSubcoreMesh(core_axis_name="c", subcore_axis_name="s")
    hist = pl.kernel(
        _sc_hist,
        out_type=jax.ShapeDtypeStruct((_NC * _HV,), jnp.float32),
        mesh=mesh,
        scratch_types=[
            pltpu.VMEM((_NIC, _IC), jnp.int32),
            pltpu.VMEM((_IC,), jnp.float32),
            pltpu.VMEM((_ZCHUNK,), jnp.float32),
            pltpu.VMEM_SHARED((_V,), jnp.float32),
        ],
    )(movies, zeros)

    tabT = movie_table.T
    userT = user_table.T
    out = pl.pallas_call(
        _tc_matvec,
        grid=(_NSTEP,),
        in_specs=[
            pl.BlockSpec((_D, _C), lambda i: (0, i)),
            pl.BlockSpec((_C,), lambda i: (i,)),
            pl.BlockSpec((_C,), lambda i: (i + _NSTEP,)),
            pl.BlockSpec((_D, 1), lambda i: (0, 0)),
        ],
        out_specs=pl.BlockSpec((_D, 1), lambda i: (0, 0)),
        out_shape=jax.ShapeDtypeStruct((_D, 1), jnp.float32),
        scratch_shapes=[pltpu.VMEM((_D, 128), jnp.float32)],
    )(tabT, hist, hist, userT)
    return out.reshape(1, _D)


def kernel(users, movies, movie_table, user_table):
    return _run(movies.astype(jnp.int32), movie_table, user_table)
